# baseline (device time: 70088 ns/iter reference)
import jax
import jax.numpy as jnp
from jax import lax
from jax.experimental import pallas as pl
from jax.experimental.pallas import tpu as pltpu

N_DEV = 16


def kernel(x, w_mat):
    m_total, k_shard = x.shape
    k_total, n_total = w_mat.shape
    m_blk = m_total // N_DEV

    def body(idx_ref, x_ref, w_ref, out_ref, xbf_ref, xg_ref,
             send_sems, recv_sems):
        t = pl.program_id(0)
        my = idx_ref[0]
        o = idx_ref[t]

        @pl.when(t == 0)
        def _():
            xbf_ref[...] = x_ref[...].astype(jnp.bfloat16)
            xg_ref[my] = xbf_ref[pl.ds(my * m_blk, m_blk), :]

        for s in range(1, N_DEV):
            j = lax.rem(my - s + N_DEV, N_DEV)
            rdma = pltpu.make_async_remote_copy(
                src_ref=xbf_ref.at[pl.ds(j * m_blk, m_blk), :],
                dst_ref=xg_ref.at[my],
                send_sem=send_sems.at[s],
                recv_sem=recv_sems.at[my],
                device_id=j,
                device_id_type=pl.DeviceIdType.LOGICAL,
            )

            @pl.when(t == 0)
            def _():
                rdma.start()

            @pl.when(t == N_DEV - 1)
            def _():
                rdma.wait_send()

        recv = pltpu.make_async_remote_copy(
            src_ref=xg_ref.at[o],
            dst_ref=xg_ref.at[o],
            send_sem=send_sems.at[0],
            recv_sem=recv_sems.at[o],
            device_id=my,
            device_id_type=pl.DeviceIdType.LOGICAL,
        )

        @pl.when(t != 0)
        def _():
            recv.wait_recv()

        prod = jnp.dot(
            xg_ref[o],
            w_ref[...].astype(jnp.bfloat16),
            preferred_element_type=jnp.float32,
        )

        @pl.when(t == 0)
        def _():
            out_ref[...] = prod

        @pl.when((t != 0) & (t != N_DEV - 1))
        def _():
            out_ref[...] = out_ref[...] + prod

        @pl.when(t == N_DEV - 1)
        def _():
            out_ref[...] = jnp.maximum(out_ref[...] + prod, 0.0)

    grid_spec = pltpu.PrefetchScalarGridSpec(
        num_scalar_prefetch=1,
        grid=(N_DEV,),
        in_specs=[
            pl.BlockSpec((m_total, k_shard), lambda t, idx: (0, 0)),
            pl.BlockSpec((k_total // N_DEV, n_total), lambda t, idx: (idx[t], 0)),
        ],
        out_specs=pl.BlockSpec((m_blk, n_total), lambda t, idx: (0, 0)),
        scratch_shapes=[
            pltpu.VMEM((m_total, k_shard), jnp.bfloat16),
            pltpu.VMEM((N_DEV, m_blk, k_shard), jnp.bfloat16),
            pltpu.SemaphoreType.DMA((N_DEV,)),
            pltpu.SemaphoreType.DMA((N_DEV,)),
        ],
    )

    idx = jnp.mod(
        lax.axis_index("i") + jnp.arange(N_DEV, dtype=jnp.int32), N_DEV
    ).astype(jnp.int32)

    return pl.pallas_call(
        body,
        grid_spec=grid_spec,
        out_shape=jax.ShapeDtypeStruct((m_blk, n_total), jnp.float32),
        compiler_params=pltpu.CompilerParams(
            dimension_semantics=("arbitrary",),
        ),
    )(idx, x, w_mat)
